# Initial kernel scaffold; baseline (speedup 1.0000x reference)
import jax
import jax.numpy as jnp
from jax.experimental import pallas as pl


def kernel(x_num, tables, W1, b1, g1, be1, W2, b2, g2, be2, W3, b3, x_cat):
    raise NotImplementedError("write your pallas kernel here")


# speedup vs baseline: 7.7341x; 7.7341x over previous
import functools

import jax
import jax.numpy as jnp
from jax import lax
from jax.experimental import pallas as pl
from jax.experimental.pallas import tpu as pltpu
from jax.experimental.pallas import tpu_sc as plsc

B = 16384
F = 26
V = 100000
E = 16
ND = 13
H1, H2 = 256, 128

_N = B * F
_NC = 2
_NS = 16
_NW = _NC * _NS
_BPW = _N // _NW
_CH = 1664
_NCHUNK = _BPW // _CH


def _build_gather():
    mesh = plsc.VectorSubcoreMesh(core_axis_name="c", subcore_axis_name="s")

    @functools.partial(
        pl.kernel,
        mesh=mesh,
        compiler_params=pltpu.CompilerParams(use_tc_tiling_on_sc=False),
        out_type=jax.ShapeDtypeStruct((_N, E), jnp.float32),
        scratch_types=[
            pltpu.VMEM((_BPW,), jnp.int32),
            pltpu.VMEM((_CH, E), jnp.float32),
            pltpu.VMEM((_CH, E), jnp.float32),
            pltpu.SemaphoreType.DMA,
            pltpu.SemaphoreType.DMA,
        ],
    )
    def gather_k(table_hbm, idx_hbm, out_hbm, idx_v, buf0, buf1, sem0, sem1):
        wid = lax.axis_index("s") * _NC + lax.axis_index("c")
        base = wid * _BPW
        pltpu.sync_copy(idx_hbm.at[pl.ds(base, _BPW)], idx_v)

        bufs = (buf0, buf1)
        sems = (sem0, sem1)

        for b in range(2):
            pltpu.async_copy(
                table_hbm.at[idx_v.at[pl.ds(b * _CH, _CH)]], bufs[b], sems[b]
            )

        @pl.loop(0, _NCHUNK, step=2)
        def _outer(g0):
            for b in range(2):
                g = g0 + b
                pltpu.make_async_copy(
                    table_hbm.at[idx_v.at[pl.ds(0, _CH)]], bufs[b], sems[b]
                ).wait()
                pltpu.sync_copy(bufs[b], out_hbm.at[pl.ds(base + g * _CH, _CH)])

                @pl.when(g + 2 < _NCHUNK)
                def _():
                    pltpu.async_copy(
                        table_hbm.at[idx_v.at[pl.ds((g + 2) * _CH, _CH)]],
                        bufs[b],
                        sems[b],
                    )

    return gather_k


_sc_gather = _build_gather()


_BB = 1024
_NB = B // _BB


def _gelu(x):
    return 0.5 * x * (1.0 + lax.erf(x * 0.7071067811865476))


def _mlp1_body(emb_ref, xn_ref, w1e_ref, w1n_ref, b1_ref, h1_ref, stats_ref,
               acc_ref):
    i = pl.program_id(0)

    @pl.when(i == 0)
    def _():
        acc_ref[...] = jnp.zeros_like(acc_ref)

    dn = (((1,), (1,)), ((), ()))
    h = (lax.dot_general(emb_ref[...], w1e_ref[...], dn,
                         preferred_element_type=jnp.float32)
         + lax.dot_general(xn_ref[...], w1n_ref[...], dn,
                           preferred_element_type=jnp.float32)
         + b1_ref[...])
    h1_ref[...] = h
    acc_ref[...] += jnp.concatenate(
        [jnp.sum(h, axis=0, keepdims=True),
         jnp.sum(h * h, axis=0, keepdims=True)], axis=0)

    @pl.when(i == _NB - 1)
    def _():
        stats_ref[...] = acc_ref[...]


def _mlp2_body(h1_ref, stats_ref, g1_ref, be1_ref, w2_ref, b2_ref, h2_ref,
               stats2_ref, acc_ref):
    i = pl.program_id(0)

    @pl.when(i == 0)
    def _():
        acc_ref[...] = jnp.zeros_like(acc_ref)

    mu = stats_ref[0:1, :] * (1.0 / B)
    var = stats_ref[1:2, :] * (1.0 / B) - mu * mu
    inv = lax.rsqrt(var + 1e-5)
    a = _gelu((h1_ref[...] - mu) * (inv * g1_ref[...]) + be1_ref[...])
    dn = (((1,), (1,)), ((), ()))
    h = (lax.dot_general(a, w2_ref[...], dn,
                         preferred_element_type=jnp.float32) + b2_ref[...])
    h2_ref[...] = h
    acc_ref[...] += jnp.concatenate(
        [jnp.sum(h, axis=0, keepdims=True),
         jnp.sum(h * h, axis=0, keepdims=True)], axis=0)

    @pl.when(i == _NB - 1)
    def _():
        stats2_ref[...] = acc_ref[...]


def _mlp3_body(h2_ref, stats2_ref, g2_ref, be2_ref, w3_ref, b3_ref, out_ref):
    mu = stats2_ref[0:1, :] * (1.0 / B)
    var = stats2_ref[1:2, :] * (1.0 / B) - mu * mu
    inv = lax.rsqrt(var + 1e-5)
    a = _gelu((h2_ref[...] - mu) * (inv * g2_ref[...]) + be2_ref[...])
    o = jnp.sum(a * w3_ref[...], axis=1, keepdims=True)
    out_ref[...] = o + b3_ref[...]


def _full(shape):
    return pl.BlockSpec(shape, lambda i: (0, 0))


_seq = pltpu.CompilerParams(dimension_semantics=("arbitrary",))


def kernel(x_num, tables, W1, b1, g1, be1, W2, b2, g2, be2, W3, b3, x_cat):
    idx_flat = (x_cat.astype(jnp.int32)
                + (jnp.arange(F, dtype=jnp.int32) * V)[None, :]).reshape(-1)
    table_flat = tables.reshape(F * V, E)
    emb = _sc_gather(table_flat, idx_flat).reshape(B, F * E)

    W1e = W1[:, :F * E]
    W1n = W1[:, F * E:]
    b1r = b1.reshape(1, H1)
    g1r = g1.reshape(1, H1)
    be1r = be1.reshape(1, H1)
    b2r = b2.reshape(1, H2)
    g2r = g2.reshape(1, H2)
    be2r = be2.reshape(1, H2)
    b3r = b3.reshape(1, 1)

    h1, stats1 = pl.pallas_call(
        _mlp1_body,
        grid=(_NB,),
        in_specs=[
            pl.BlockSpec((_BB, F * E), lambda i: (i, 0)),
            pl.BlockSpec((_BB, ND), lambda i: (i, 0)),
            _full((H1, F * E)),
            _full((H1, ND)),
            _full((1, H1)),
        ],
        out_specs=[
            pl.BlockSpec((_BB, H1), lambda i: (i, 0)),
            _full((2, H1)),
        ],
        out_shape=[
            jax.ShapeDtypeStruct((B, H1), jnp.float32),
            jax.ShapeDtypeStruct((2, H1), jnp.float32),
        ],
        scratch_shapes=[pltpu.VMEM((2, H1), jnp.float32)],
        compiler_params=_seq,
    )(emb, x_num, W1e, W1n, b1r)

    h2, stats2 = pl.pallas_call(
        _mlp2_body,
        grid=(_NB,),
        in_specs=[
            pl.BlockSpec((_BB, H1), lambda i: (i, 0)),
            _full((2, H1)),
            _full((1, H1)),
            _full((1, H1)),
            _full((H2, H1)),
            _full((1, H2)),
        ],
        out_specs=[
            pl.BlockSpec((_BB, H2), lambda i: (i, 0)),
            _full((2, H2)),
        ],
        out_shape=[
            jax.ShapeDtypeStruct((B, H2), jnp.float32),
            jax.ShapeDtypeStruct((2, H2), jnp.float32),
        ],
        scratch_shapes=[pltpu.VMEM((2, H2), jnp.float32)],
        compiler_params=_seq,
    )(h1, stats1, g1r, be1r, W2, b2r)

    out = pl.pallas_call(
        _mlp3_body,
        grid=(_NB,),
        in_specs=[
            pl.BlockSpec((_BB, H2), lambda i: (i, 0)),
            _full((2, H2)),
            _full((1, H2)),
            _full((1, H2)),
            _full((1, H2)),
            _full((1, 1)),
        ],
        out_specs=pl.BlockSpec((_BB, 1), lambda i: (i, 0)),
        out_shape=jax.ShapeDtypeStruct((B, 1), jnp.float32),
        compiler_params=_seq,
    )(h2, stats2, g2r, be2r, W3, b3r)

    return out.reshape(B)
